# baseline (speedup 1.0000x reference)
import functools

import jax
import jax.numpy as jnp
from jax import lax
from jax.experimental import pallas as pl
from jax.experimental.pallas import tpu as pltpu
from jax.experimental.pallas import tpu_sc as plsc

NUM_EMB = 1000000
DIM = 64
PDIM = 128
BATCH = 4096
FIELDS = 26
TOTAL = BATCH * FIELDS

NC = 2
NS = 16
NW = NC * NS

SLAB = 256
IDS_W = 31232
NSLAB = IDS_W // SLAB
TAIL0 = NW * IDS_W

PER_W = TOTAL // NW
CHUNK = 128
NCHUNK = PER_W // CHUNK
NBUF = 4

_mesh = plsc.VectorSubcoreMesh(core_axis_name="c", subcore_axis_name="s")


@functools.partial(
    pl.kernel,
    mesh=_mesh,
    out_type=jax.ShapeDtypeStruct((NUM_EMB, PDIM), jnp.float32),
    compiler_params=pltpu.CompilerParams(
        use_tc_tiling_on_sc=True, needs_layout_passes=False),
    scratch_types=[
        pltpu.VMEM((2, 8, 8, SLAB), jnp.float32),
        pltpu.VMEM((2, SLAB, PDIM), jnp.float32),
        pltpu.VMEM((32, 128), jnp.float32),
        pltpu.SemaphoreType.DMA,
        pltpu.SemaphoreType.DMA,
    ],
)
def _transpose_kernel(wT_hbm, tail_hbm, out_hbm, in_v, stage_v, tail_v,
                      lsem, wsem):
    wid = lax.axis_index("s") * NC + lax.axis_index("c")
    base = wid * IDS_W

    def load_slab(slot, id0, width):
        for g in range(8):
            dst = in_v.at[slot, g]
            if width != SLAB:
                dst = dst.at[:, pl.ds(0, width)]
            pltpu.async_copy(
                wT_hbm.at[pl.ds(8 * g, 8), pl.ds(id0, width)], dst, lsem)

    def wait_slab(slot, width):
        for g in range(8):
            dst = in_v.at[slot, g]
            if width != SLAB:
                dst = dst.at[:, pl.ds(0, width)]
            pltpu.make_async_copy(
                wT_hbm.at[pl.ds(0, 8), pl.ds(0, width)], dst, lsem).wait()

    def shuffle(slot, width):
        iota = lax.iota(jnp.int32, 16)

        def body(iq, carry):
            for u in range(4):
                i0 = iq * 4 + u
                rows = iota + i0 * 16
                for g in range(8):
                    for c_lo in range(8):
                        c = 8 * g + c_lo
                        val = in_v[slot, g, c_lo, pl.ds(i0 * 16, 16)]
                        plsc.store_scatter(
                            stage_v.at[slot],
                            [rows, jnp.full((16,), c, jnp.int32)], val)
            return carry

        lax.fori_loop(0, width // 64, body, 0)

    load_slab(0, base, SLAB)

    def outer(s2, carry):
        for slot in range(2):
            s = s2 * 2 + slot
            wait_slab(slot, SLAB)

            @pl.when(s + 1 < NSLAB)
            def _():
                load_slab(1 - slot, base + (s + 1) * SLAB, SLAB)

            shuffle(slot, SLAB)

            @pl.when(s >= 1)
            def _():
                pltpu.make_async_copy(
                    stage_v.at[1 - slot],
                    out_hbm.at[pl.ds(0, SLAB)], wsem).wait()

            pltpu.async_copy(
                stage_v.at[slot],
                out_hbm.at[pl.ds(base + s * SLAB, SLAB)], wsem)
        return carry

    lax.fori_loop(0, NSLAB // 2, outer, 0)
    pltpu.make_async_copy(
        stage_v.at[1], out_hbm.at[pl.ds(0, SLAB)], wsem).wait()

    @pl.when(wid < 2)
    def _():
        t0 = TAIL0 + wid * SLAB
        load_slab(0, t0, SLAB)
        wait_slab(0, SLAB)
        shuffle(0, SLAB)
        pltpu.async_copy(stage_v.at[0], out_hbm.at[pl.ds(t0, SLAB)], wsem)
        pltpu.make_async_copy(
            stage_v.at[0], out_hbm.at[pl.ds(t0, SLAB)], wsem).wait()

    @pl.when(wid == 2)
    def _():
        t0 = TAIL0 + 2 * SLAB
        pltpu.sync_copy(tail_hbm, tail_v)
        for k in range(64):
            for q in range(4):
                flat = k * 64 + q * 16
                stage_v[0, k, pl.ds(q * 16, 16)] = (
                    tail_v[flat // 128, pl.ds(flat % 128, 16)])
        pltpu.async_copy(
            stage_v.at[0].at[pl.ds(0, 64)], out_hbm.at[pl.ds(t0, 64)], wsem)
        pltpu.make_async_copy(
            stage_v.at[0].at[pl.ds(0, 64)], out_hbm.at[pl.ds(t0, 64)],
            wsem).wait()


@functools.partial(
    pl.kernel,
    mesh=_mesh,
    out_type=jax.ShapeDtypeStruct((TOTAL, PDIM), jnp.float32),
    compiler_params=pltpu.CompilerParams(use_tc_tiling_on_sc=True),
    scratch_types=[
        pltpu.VMEM((PER_W,), jnp.int32),
        pltpu.VMEM((NBUF, CHUNK, PDIM), jnp.float32),
        pltpu.SemaphoreType.DMA,
        pltpu.SemaphoreType.DMA,
    ],
)
def _gather_kernel(idx_hbm, tab_hbm, out_hbm, idx_v, rows_v, gsem, wsem):
    wid = lax.axis_index("s") * NC + lax.axis_index("c")
    base = wid * PER_W

    pltpu.sync_copy(idx_hbm.at[pl.ds(base, PER_W)], idx_v)

    gathers = [None] * NCHUNK
    writes = [None] * NCHUNK
    for j in range(min(NBUF, NCHUNK)):
        gathers[j] = pltpu.async_copy(
            tab_hbm.at[idx_v.at[pl.ds(j * CHUNK, CHUNK)]],
            rows_v.at[j % NBUF], gsem)
    for j in range(NCHUNK):
        gathers[j].wait()
        writes[j] = pltpu.async_copy(
            rows_v.at[j % NBUF],
            out_hbm.at[pl.ds(base + j * CHUNK, CHUNK)],
            wsem)
        nxt = j + NBUF
        if nxt < NCHUNK:
            writes[j].wait()
            gathers[nxt] = pltpu.async_copy(
                tab_hbm.at[idx_v.at[pl.ds(nxt * CHUNK, CHUNK)]],
                rows_v.at[nxt % NBUF], gsem)
    for j in range(max(0, NCHUNK - NBUF), NCHUNK):
        writes[j].wait()


def kernel(x, weight):
    idx = x.astype(jnp.int32).reshape(TOTAL)
    tail = weight[TAIL0 + 2 * SLAB:, :].reshape(32, 128)
    wlin = _transpose_kernel(weight.T, tail)
    out = _gather_kernel(idx, wlin)
    return out[:, :DIM].reshape(BATCH, FIELDS, DIM)


# speedup vs baseline: 1.6806x; 1.6806x over previous
import functools

import jax
import jax.numpy as jnp
from jax import lax
from jax.experimental import pallas as pl
from jax.experimental.pallas import tpu as pltpu
from jax.experimental.pallas import tpu_sc as plsc

NUM_EMB = 1000000
DIM = 64
PDIM = 128
BATCH = 4096
FIELDS = 26
TOTAL = BATCH * FIELDS

NC = 2
NS = 16
NW = NC * NS

SLAB = 256
IDS_W = 31232
NSLAB = IDS_W // SLAB
TAIL0 = NW * IDS_W

PER_W = TOTAL // NW
CHUNK = 128
NCHUNK = PER_W // CHUNK
NBUF = 4

_mesh = plsc.VectorSubcoreMesh(core_axis_name="c", subcore_axis_name="s")


@functools.partial(
    pl.kernel,
    mesh=_mesh,
    out_type=jax.ShapeDtypeStruct((NUM_EMB, PDIM), jnp.float32),
    compiler_params=pltpu.CompilerParams(
        use_tc_tiling_on_sc=True, needs_layout_passes=False),
    scratch_types=[
        pltpu.VMEM((2, 8, 8, SLAB), jnp.float32),
        pltpu.VMEM((2, SLAB, PDIM), jnp.float32),
        pltpu.VMEM((32, 128), jnp.float32),
        pltpu.SemaphoreType.DMA,
        pltpu.SemaphoreType.DMA,
    ],
)
def _transpose_kernel(wT_hbm, tail_hbm, out_hbm, in_v, stage_v, tail_v,
                      lsem, wsem):
    wid = lax.axis_index("s") * NC + lax.axis_index("c")
    base = wid * IDS_W

    def load_slab(slot, id0, width):
        for g in range(8):
            dst = in_v.at[slot, g]
            if width != SLAB:
                dst = dst.at[:, pl.ds(0, width)]
            pltpu.async_copy(
                wT_hbm.at[pl.ds(8 * g, 8), pl.ds(id0, width)], dst, lsem)

    def wait_slab(slot, width):
        for g in range(8):
            dst = in_v.at[slot, g]
            if width != SLAB:
                dst = dst.at[:, pl.ds(0, width)]
            pltpu.make_async_copy(
                wT_hbm.at[pl.ds(0, 8), pl.ds(0, width)], dst, lsem).wait()

    def shuffle(slot, width):
        iota = lax.iota(jnp.int32, 16)
        src = in_v.at[slot]
        dst = stage_v.at[slot]

        def body(ib, carry):
            rows = iota + ib * 16
            for c0 in range(0, 64, 16):
                for k in range(16):
                    c_abs = c0 + ((iota + k) & 15)
                    val = plsc.load_gather(
                        src, [c_abs >> 3, c_abs & 7, rows])
                    plsc.store_scatter(dst, [rows, c_abs], val)
            return carry

        lax.fori_loop(0, width // 16, body, 0)

    load_slab(0, base, SLAB)

    def outer(s2, carry):
        for slot in range(2):
            s = s2 * 2 + slot
            wait_slab(slot, SLAB)

            @pl.when(s + 1 < NSLAB)
            def _():
                load_slab(1 - slot, base + (s + 1) * SLAB, SLAB)

            shuffle(slot, SLAB)

            @pl.when(s >= 1)
            def _():
                pltpu.make_async_copy(
                    stage_v.at[1 - slot],
                    out_hbm.at[pl.ds(0, SLAB)], wsem).wait()

            pltpu.async_copy(
                stage_v.at[slot],
                out_hbm.at[pl.ds(base + s * SLAB, SLAB)], wsem)
        return carry

    lax.fori_loop(0, NSLAB // 2, outer, 0)
    pltpu.make_async_copy(
        stage_v.at[1], out_hbm.at[pl.ds(0, SLAB)], wsem).wait()

    @pl.when(wid < 2)
    def _():
        t0 = TAIL0 + wid * SLAB
        load_slab(0, t0, SLAB)
        wait_slab(0, SLAB)
        shuffle(0, SLAB)
        pltpu.async_copy(stage_v.at[0], out_hbm.at[pl.ds(t0, SLAB)], wsem)
        pltpu.make_async_copy(
            stage_v.at[0], out_hbm.at[pl.ds(t0, SLAB)], wsem).wait()

    @pl.when(wid == 2)
    def _():
        t0 = TAIL0 + 2 * SLAB
        pltpu.sync_copy(tail_hbm, tail_v)
        for k in range(64):
            for q in range(4):
                flat = k * 64 + q * 16
                stage_v[0, k, pl.ds(q * 16, 16)] = (
                    tail_v[flat // 128, pl.ds(flat % 128, 16)])
        pltpu.async_copy(
            stage_v.at[0].at[pl.ds(0, 64)], out_hbm.at[pl.ds(t0, 64)], wsem)
        pltpu.make_async_copy(
            stage_v.at[0].at[pl.ds(0, 64)], out_hbm.at[pl.ds(t0, 64)],
            wsem).wait()


@functools.partial(
    pl.kernel,
    mesh=_mesh,
    out_type=jax.ShapeDtypeStruct((TOTAL, PDIM), jnp.float32),
    compiler_params=pltpu.CompilerParams(use_tc_tiling_on_sc=True),
    scratch_types=[
        pltpu.VMEM((PER_W,), jnp.int32),
        pltpu.VMEM((NBUF, CHUNK, PDIM), jnp.float32),
        pltpu.SemaphoreType.DMA,
        pltpu.SemaphoreType.DMA,
    ],
)
def _gather_kernel(idx_hbm, tab_hbm, out_hbm, idx_v, rows_v, gsem, wsem):
    wid = lax.axis_index("s") * NC + lax.axis_index("c")
    base = wid * PER_W

    pltpu.sync_copy(idx_hbm.at[pl.ds(base, PER_W)], idx_v)

    gathers = [None] * NCHUNK
    writes = [None] * NCHUNK
    for j in range(min(NBUF, NCHUNK)):
        gathers[j] = pltpu.async_copy(
            tab_hbm.at[idx_v.at[pl.ds(j * CHUNK, CHUNK)]],
            rows_v.at[j % NBUF], gsem)
    for j in range(NCHUNK):
        gathers[j].wait()
        writes[j] = pltpu.async_copy(
            rows_v.at[j % NBUF],
            out_hbm.at[pl.ds(base + j * CHUNK, CHUNK)],
            wsem)
        nxt = j + NBUF
        if nxt < NCHUNK:
            writes[j].wait()
            gathers[nxt] = pltpu.async_copy(
                tab_hbm.at[idx_v.at[pl.ds(nxt * CHUNK, CHUNK)]],
                rows_v.at[nxt % NBUF], gsem)
    for j in range(max(0, NCHUNK - NBUF), NCHUNK):
        writes[j].wait()


def kernel(x, weight):
    idx = x.astype(jnp.int32).reshape(TOTAL)
    tail = weight[TAIL0 + 2 * SLAB:, :].reshape(32, 128)
    wlin = _transpose_kernel(weight.T, tail)
    out = _gather_kernel(idx, wlin)
    return out[:, :DIM].reshape(BATCH, FIELDS, DIM)


# speedup vs baseline: 3.1260x; 1.8601x over previous
import functools

import jax
import jax.numpy as jnp
from jax import lax
from jax.experimental import pallas as pl
from jax.experimental.pallas import tpu as pltpu
from jax.experimental.pallas import tpu_sc as plsc

NUM_EMB = 1000000
DIM = 64
PDIM = 128
BATCH = 4096
FIELDS = 26
TOTAL = BATCH * FIELDS

NC = 2
NS = 16
NW = NC * NS

SLAB = 256
IDS_W = 31232
NSLAB = IDS_W // SLAB
TAIL0 = NW * IDS_W

PER_W = TOTAL // NW
CHUNK = 128
NCHUNK = PER_W // CHUNK
NBUF = 4

_mesh = plsc.VectorSubcoreMesh(core_axis_name="c", subcore_axis_name="s")


@functools.partial(
    pl.kernel,
    mesh=_mesh,
    out_type=jax.ShapeDtypeStruct((NUM_EMB, PDIM), jnp.float32),
    compiler_params=pltpu.CompilerParams(
        use_tc_tiling_on_sc=True, needs_layout_passes=False),
    scratch_types=[
        pltpu.VMEM((2, 8, 8, SLAB), jnp.float32),
        pltpu.VMEM((2, SLAB, PDIM), jnp.float32),
        pltpu.VMEM((32, 128), jnp.float32),
        pltpu.SemaphoreType.DMA,
        pltpu.SemaphoreType.DMA,
    ],
)
def _transpose_kernel(wT_hbm, tail_hbm, out_hbm, in_v, stage_v, tail_v,
                      lsem, wsem):
    wid = lax.axis_index("s") * NC + lax.axis_index("c")
    base = wid * IDS_W

    def load_slab(slot, id0, width):
        for g in range(8):
            dst = in_v.at[slot, g]
            if width != SLAB:
                dst = dst.at[:, pl.ds(0, width)]
            pltpu.async_copy(
                wT_hbm.at[pl.ds(8 * g, 8), pl.ds(id0, width)], dst, lsem)

    def wait_slab(slot, width):
        for g in range(8):
            dst = in_v.at[slot, g]
            if width != SLAB:
                dst = dst.at[:, pl.ds(0, width)]
            pltpu.make_async_copy(
                wT_hbm.at[pl.ds(0, 8), pl.ds(0, width)], dst, lsem).wait()

    def shuffle(slot, width):
        iota = lax.iota(jnp.int32, 16)
        src = in_v.at[slot]
        dst = stage_v.at[slot]

        def body(ib, carry):
            rows = iota + ib * 16
            for k in range(16):
                c_rel = (iota + k) & 15
                vals = []
                for c0 in range(0, 64, 16):
                    c_abs = c0 + c_rel
                    vals.append((c_abs, plsc.load_gather(
                        src, [c_abs >> 3, c_abs & 7, rows])))
                for c_abs, val in vals:
                    plsc.store_scatter(dst, [rows, c_abs], val)
            return carry

        lax.fori_loop(0, width // 16, body, 0)

    load_slab(0, base, SLAB)

    def outer(s2, carry):
        for slot in range(2):
            s = s2 * 2 + slot
            wait_slab(slot, SLAB)

            @pl.when(s + 1 < NSLAB)
            def _():
                load_slab(1 - slot, base + (s + 1) * SLAB, SLAB)

            shuffle(slot, SLAB)

            @pl.when(s >= 1)
            def _():
                pltpu.make_async_copy(
                    stage_v.at[1 - slot],
                    out_hbm.at[pl.ds(0, SLAB)], wsem).wait()

            pltpu.async_copy(
                stage_v.at[slot],
                out_hbm.at[pl.ds(base + s * SLAB, SLAB)], wsem)
        return carry

    lax.fori_loop(0, NSLAB // 2, outer, 0)
    pltpu.make_async_copy(
        stage_v.at[1], out_hbm.at[pl.ds(0, SLAB)], wsem).wait()

    @pl.when(wid < 2)
    def _():
        t0 = TAIL0 + wid * SLAB
        load_slab(0, t0, SLAB)
        wait_slab(0, SLAB)
        shuffle(0, SLAB)
        pltpu.async_copy(stage_v.at[0], out_hbm.at[pl.ds(t0, SLAB)], wsem)
        pltpu.make_async_copy(
            stage_v.at[0], out_hbm.at[pl.ds(t0, SLAB)], wsem).wait()

    @pl.when(wid == 2)
    def _():
        t0 = TAIL0 + 2 * SLAB
        pltpu.sync_copy(tail_hbm, tail_v)
        for k in range(64):
            for q in range(4):
                flat = k * 64 + q * 16
                stage_v[0, k, pl.ds(q * 16, 16)] = (
                    tail_v[flat // 128, pl.ds(flat % 128, 16)])
        pltpu.async_copy(
            stage_v.at[0].at[pl.ds(0, 64)], out_hbm.at[pl.ds(t0, 64)], wsem)
        pltpu.make_async_copy(
            stage_v.at[0].at[pl.ds(0, 64)], out_hbm.at[pl.ds(t0, 64)],
            wsem).wait()


@functools.partial(
    pl.kernel,
    mesh=_mesh,
    out_type=jax.ShapeDtypeStruct((TOTAL, PDIM), jnp.float32),
    compiler_params=pltpu.CompilerParams(use_tc_tiling_on_sc=True),
    scratch_types=[
        pltpu.VMEM((PER_W,), jnp.int32),
        pltpu.VMEM((NBUF, CHUNK, PDIM), jnp.float32),
        pltpu.SemaphoreType.DMA,
        pltpu.SemaphoreType.DMA,
    ],
)
def _gather_kernel(idx_hbm, tab_hbm, out_hbm, idx_v, rows_v, gsem, wsem):
    wid = lax.axis_index("s") * NC + lax.axis_index("c")
    base = wid * PER_W

    pltpu.sync_copy(idx_hbm.at[pl.ds(base, PER_W)], idx_v)

    gathers = [None] * NCHUNK
    writes = [None] * NCHUNK
    for j in range(min(NBUF, NCHUNK)):
        gathers[j] = pltpu.async_copy(
            tab_hbm.at[idx_v.at[pl.ds(j * CHUNK, CHUNK)]],
            rows_v.at[j % NBUF], gsem)
    for j in range(NCHUNK):
        gathers[j].wait()
        writes[j] = pltpu.async_copy(
            rows_v.at[j % NBUF],
            out_hbm.at[pl.ds(base + j * CHUNK, CHUNK)],
            wsem)
        nxt = j + NBUF
        if nxt < NCHUNK:
            writes[j].wait()
            gathers[nxt] = pltpu.async_copy(
                tab_hbm.at[idx_v.at[pl.ds(nxt * CHUNK, CHUNK)]],
                rows_v.at[nxt % NBUF], gsem)
    for j in range(max(0, NCHUNK - NBUF), NCHUNK):
        writes[j].wait()


def kernel(x, weight):
    idx = x.astype(jnp.int32).reshape(TOTAL)
    tail = weight[TAIL0 + 2 * SLAB:, :].reshape(32, 128)
    wlin = _transpose_kernel(weight.T, tail)
    out = _gather_kernel(idx, wlin)
    return out[:, :DIM].reshape(BATCH, FIELDS, DIM)
